# baseline (speedup 1.0000x reference)
import jax
import jax.numpy as jnp
from jax import lax
from jax.experimental import pallas as pl
from jax.experimental.pallas import tpu as pltpu
from jax.experimental.pallas import tpu_sc as plsc

_C = 3
_F = 5
_K = _F * _F
_HO = 48
_WO = 48
_HI = _HO + _F - 1
_WI = _WO + _F - 1
_RPT = 2
_NTILES = _HO // _RPT
_PPT = _RPT * _WO
_NVEC = _PPT // 16


def _dsep_body(inp_hbm, vt_hbm, ht_hbm, ox_hbm, oy_hbm, mk_hbm, out_hbm,
               inp_v, vt_v, ht_v, ox_v, oy_v, mk_v, out_v, sem):
  wid = lax.axis_index("s") * 2 + lax.axis_index("c")

  @pl.when(wid < _NTILES)
  def _():
    row0 = wid * _RPT
    rows = pl.ds(row0, _RPT)

    copies = [
        pltpu.async_copy(inp_hbm.at[0], inp_v, sem),
        pltpu.async_copy(vt_hbm.at[0, :, rows, :], vt_v, sem),
        pltpu.async_copy(ht_hbm.at[0, :, rows, :], ht_v, sem),
        pltpu.async_copy(ox_hbm.at[0, :, rows, :], ox_v, sem),
        pltpu.async_copy(oy_hbm.at[0, :, rows, :], oy_v, sem),
        pltpu.async_copy(mk_hbm.at[0, :, rows, :], mk_v, sem),
    ]
    for cp in copies:
      cp.wait()

    iota = lax.broadcasted_iota(jnp.int32, (16,), 0)
    zero = jnp.zeros((16,), jnp.float32)
    zero_i = jnp.zeros((16,), jnp.int32)
    row0_f = row0.astype(jnp.float32)

    for vec in range(_NVEC):
      r_v, c_v = divmod(vec * 16, _WO)
      cols16 = pl.ds(c_v, 16)
      w_f = (iota + c_v).astype(jnp.float32)
      xb = [w_f + float(fx - 1) for fx in range(_F)]
      yb = [row0_f + float(r_v + fy - 1) for fy in range(_F)]
      vv = [vt_v[fy, r_v, cols16] for fy in range(_F)]
      hh = [ht_v[fx, r_v, cols16] for fx in range(_F)]
      vh = [[vv[fy] * hh[fx] for fx in range(_F)] for fy in range(_F)]

      accs = [[zero, zero] for _ in range(_C)]
      for fy in range(_F):
        for fx in range(_F):
          k = fy * _F + fx
          ox = ox_v[k, r_v, cols16]
          oy = oy_v[k, r_v, cols16]
          mk = mk_v[k, r_v, cols16]
          pos_x = jnp.minimum(jnp.maximum(oy + xb[fx], 0.0), float(_WI - 1))
          pos_y = jnp.minimum(jnp.maximum(ox + yb[fy], 0.0), float(_HI - 1))
          left = pos_x.astype(jnp.int32)
          top = pos_y.astype(jnp.int32)
          fracx = pos_x - left.astype(jnp.float32)
          fracy = pos_y - top.astype(jnp.float32)
          right = jnp.minimum(left + 1, _WI - 1)
          bot = jnp.minimum(top + 1, _HI - 1)
          wgt = vh[fy][fx] * mk
          par = fy & 1
          for c in range(_C):
            cv = zero_i + c
            tl = plsc.load_gather(inp_v, [cv, top, left])
            tr = plsc.load_gather(inp_v, [cv, top, right])
            bl = plsc.load_gather(inp_v, [cv, bot, left])
            br = plsc.load_gather(inp_v, [cv, bot, right])
            top_l = tl + fracx * (tr - tl)
            bot_l = bl + fracx * (br - bl)
            val = top_l + fracy * (bot_l - top_l)
            accs[c][par] = accs[c][par] + val * wgt
      for c in range(_C):
        out_v[c, r_v, cols16] = accs[c][0] + accs[c][1]

    for c in range(_C):
      pltpu.sync_copy(out_v.at[c], out_hbm.at[0, c, rows, :])


@jax.jit
def kernel(tensorInput, tensorVertical, tensorHorizontal, tensorOffsetX,
           tensorOffsetY, tensorMask):
  mesh = plsc.VectorSubcoreMesh(core_axis_name="c", subcore_axis_name="s")
  run = pl.kernel(
      _dsep_body,
      out_type=jax.ShapeDtypeStruct((1, _C, _HO, _WO), jnp.float32),
      mesh=mesh,
      compiler_params=pltpu.CompilerParams(
          needs_layout_passes=False, use_tc_tiling_on_sc=False),
      scratch_types=[
          pltpu.VMEM((_C, _HI, _WI), jnp.float32),
          pltpu.VMEM((_F, _RPT, _WO), jnp.float32),
          pltpu.VMEM((_F, _RPT, _WO), jnp.float32),
          pltpu.VMEM((_K, _RPT, _WO), jnp.float32),
          pltpu.VMEM((_K, _RPT, _WO), jnp.float32),
          pltpu.VMEM((_K, _RPT, _WO), jnp.float32),
          pltpu.VMEM((_C, _RPT, _WO), jnp.float32),
          pltpu.SemaphoreType.DMA,
      ],
  )
  return run(tensorInput, tensorVertical, tensorHorizontal, tensorOffsetX,
             tensorOffsetY, tensorMask)


# speedup vs baseline: 1.2014x; 1.2014x over previous
import jax
import jax.numpy as jnp
from jax import lax
from jax.experimental import pallas as pl
from jax.experimental.pallas import tpu as pltpu
from jax.experimental.pallas import tpu_sc as plsc

_C = 3
_F = 5
_K = _F * _F
_HO = 48
_WO = 48
_HI = _HO + _F - 1
_WI = _WO + _F - 1
_NPIX = _HO * _WO
_NWORKERS = 32
_PPW = _NPIX // _NWORKERS
_NVEC = (_PPW + 15) // 16
_ROX = 0
_ROY = _K
_RMK = 2 * _K
_RVT = 3 * _K
_RHT = 3 * _K + _F
_NSTK = 3 * _K + 2 * _F


def _dsep_body(stk_hbm, inp_hbm, out_hbm, stk_v, inp_v, out_v, sem):
  wid = lax.axis_index("s") * 2 + lax.axis_index("c")
  base = wid * _PPW

  copies = [
      pltpu.async_copy(stk_hbm.at[:, pl.ds(base, _PPW)], stk_v, sem),
      pltpu.async_copy(inp_hbm.at[0], inp_v, sem),
  ]
  for cp in copies:
    cp.wait()

  iota = lax.broadcasted_iota(jnp.int32, (16,), 0)
  zero = jnp.zeros((16,), jnp.float32)
  zero_i = jnp.zeros((16,), jnp.int32)

  for vec in range(_NVEC):
    lp = iota + (vec * 16)
    if (vec + 1) * 16 > _PPW:
      lp = jnp.minimum(lp, _PPW - 1)
    pix = lp + base
    h = lax.div(pix, _WO)
    w = pix - h * _WO
    h_f = h.astype(jnp.float32)
    w_f = w.astype(jnp.float32)
    hh_c = [plsc.load_gather(stk_v, [zero_i + (_RHT + fx), lp])
            for fx in range(_F)]

    def body(fy, accs, lp=lp, h_f=h_f, w_f=w_f, hh_c=hh_c):
      a0, a1, a2 = accs
      fy_vec = zero_i + fy
      vv = plsc.load_gather(stk_v, [fy_vec + _RVT, lp])
      fy_f = fy.astype(jnp.float32)
      for fx in range(_F):
        k_vec = fy_vec * _F + fx
        ox = plsc.load_gather(stk_v, [k_vec + _ROX, lp])
        oy = plsc.load_gather(stk_v, [k_vec + _ROY, lp])
        mk = plsc.load_gather(stk_v, [k_vec + _RMK, lp])
        pos_x = oy + (w_f + float(fx - 1))
        pos_y = ox + (h_f + (fy_f - 1.0))
        pos_x = jnp.minimum(jnp.maximum(pos_x, 0.0), float(_WI - 1))
        pos_y = jnp.minimum(jnp.maximum(pos_y, 0.0), float(_HI - 1))
        left = pos_x.astype(jnp.int32)
        top = pos_y.astype(jnp.int32)
        fracx = pos_x - left.astype(jnp.float32)
        fracy = pos_y - top.astype(jnp.float32)
        row_t = top * _WI
        row_b = jnp.minimum(row_t + _WI, (_HI - 1) * _WI)
        i_tl = row_t + left
        i_tr = jnp.minimum(i_tl + 1, row_t + (_WI - 1))
        dx = i_tr - i_tl
        i_bl = row_b + left
        i_br = i_bl + dx
        wgt = vv * hh_c[fx] * mk
        outs = []
        for acc, off in zip((a0, a1, a2), (0, _HI * _WI, 2 * _HI * _WI)):
          tl = plsc.load_gather(inp_v, [i_tl + off])
          tr = plsc.load_gather(inp_v, [i_tr + off])
          bl = plsc.load_gather(inp_v, [i_bl + off])
          br = plsc.load_gather(inp_v, [i_br + off])
          top_l = tl + fracx * (tr - tl)
          bot_l = bl + fracx * (br - bl)
          val = top_l + fracy * (bot_l - top_l)
          outs.append(acc + val * wgt)
        a0, a1, a2 = outs
      return a0, a1, a2

    a0, a1, a2 = lax.fori_loop(0, _F, body, (zero, zero, zero))
    out_v[pl.ds(0 * 80 + vec * 16, 16)] = a0
    out_v[pl.ds(1 * 80 + vec * 16, 16)] = a1
    out_v[pl.ds(2 * 80 + vec * 16, 16)] = a2

  for c in range(_C):
    pltpu.sync_copy(out_v.at[pl.ds(c * 80, _PPW)],
                    out_hbm.at[c, pl.ds(base, _PPW)])


@jax.jit
def _dsepconv_sc(stk, inp):
  mesh = plsc.VectorSubcoreMesh(core_axis_name="c", subcore_axis_name="s")
  run = pl.kernel(
      _dsep_body,
      out_type=jax.ShapeDtypeStruct((_C, _NPIX), jnp.float32),
      mesh=mesh,
      compiler_params=pltpu.CompilerParams(
          needs_layout_passes=False, use_tc_tiling_on_sc=False),
      scratch_types=[
          pltpu.VMEM((_NSTK, _PPW), jnp.float32),
          pltpu.VMEM((_C * _HI * _WI,), jnp.float32),
          pltpu.VMEM((_C * 80,), jnp.float32),
          pltpu.SemaphoreType.DMA,
      ],
  )
  return run(stk, inp)


def kernel(tensorInput, tensorVertical, tensorHorizontal, tensorOffsetX,
           tensorOffsetY, tensorMask):
  stk = jnp.concatenate([
      tensorOffsetX.reshape(_K, _NPIX),
      tensorOffsetY.reshape(_K, _NPIX),
      tensorMask.reshape(_K, _NPIX),
      tensorVertical.reshape(_F, _NPIX),
      tensorHorizontal.reshape(_F, _NPIX),
  ], axis=0)
  inp = tensorInput.reshape(1, _C * _HI * _WI)
  out = _dsepconv_sc(stk, inp)
  return out.reshape(1, _C, _HO, _WO)
